# baseline (speedup 1.0000x reference)
import jax
import jax.numpy as jnp
from jax.experimental import pallas as pl
from jax.experimental.pallas import tpu as pltpu

TOKEN_BLOCK = 512
ROWS = TOKEN_BLOCK * 32


def _body(x_ref, out_ref):
    out_ref[...] = x_ref[:TOKEN_BLOCK, :64]


@jax.jit
def kernel(x, router_weight):
    tokens, dim = x.shape
    num_experts = router_weight.shape[0]
    xr = x.reshape(tokens * (dim // 128), 128)

    grid = (tokens // TOKEN_BLOCK,)
    return pl.pallas_call(
        _body,
        grid=grid,
        in_specs=[
            pl.BlockSpec((ROWS, 128), lambda i: (i, 0)),
        ],
        out_specs=pl.BlockSpec((TOKEN_BLOCK, num_experts), lambda i: (i, 0)),
        out_shape=jax.ShapeDtypeStruct((tokens, num_experts), jnp.float32),
        compiler_params=pltpu.CompilerParams(
            dimension_semantics=("parallel",),
        ),
    )(xr)


# speedup vs baseline: 4.2361x; 4.2361x over previous
import jax
import jax.numpy as jnp
from jax.experimental import pallas as pl
from jax.experimental.pallas import tpu as pltpu

TOKEN_BLOCK = 512


def _router_block(x_ref, w_ref, out_ref):
    i = pl.program_id(0)
    logits = jnp.dot(x_ref[...], w_ref[...], preferred_element_type=jnp.float32)
    out_ref[pl.ds(i * TOKEN_BLOCK, TOKEN_BLOCK), :] = jax.nn.sigmoid(logits)


@jax.jit
def kernel(x, router_weight):
    tokens, dim = x.shape
    num_experts = router_weight.shape[0]
    wt = router_weight.T

    grid = (tokens // TOKEN_BLOCK,)
    return pl.pallas_call(
        _router_block,
        grid=grid,
        in_specs=[
            pl.BlockSpec((TOKEN_BLOCK, dim), lambda i: (i, 0)),
            pl.BlockSpec((dim, num_experts), lambda i: (0, 0)),
        ],
        out_specs=pl.BlockSpec((tokens, num_experts), lambda i: (0, 0)),
        out_shape=jax.ShapeDtypeStruct((tokens, num_experts), jnp.float32),
        compiler_params=pltpu.CompilerParams(
            dimension_semantics=("arbitrary",),
        ),
    )(x, wt)


# speedup vs baseline: 4.2751x; 1.0092x over previous
import jax
import jax.numpy as jnp
from jax.experimental import pallas as pl
from jax.experimental.pallas import tpu as pltpu

TOKEN_BLOCK = 512


def _router_block(x_ref, w_ref, out_ref):
    xh = x_ref[...].astype(jnp.bfloat16)
    logits = jnp.dot(xh, w_ref[...], preferred_element_type=jnp.float32)
    out_ref[...] = jax.nn.sigmoid(logits)


@jax.jit
def kernel(x, router_weight):
    tokens, dim = x.shape
    num_experts = router_weight.shape[0]
    wt = router_weight.T.astype(jnp.bfloat16)

    grid = (tokens // TOKEN_BLOCK,)
    return pl.pallas_call(
        _router_block,
        grid=grid,
        in_specs=[
            pl.BlockSpec((TOKEN_BLOCK, dim), lambda i: (i, 0)),
            pl.BlockSpec((dim, num_experts), lambda i: (0, 0)),
        ],
        out_specs=pl.BlockSpec((TOKEN_BLOCK, num_experts), lambda i: (i, 0)),
        out_shape=jax.ShapeDtypeStruct((tokens, num_experts), jnp.float32),
        compiler_params=pltpu.CompilerParams(
            dimension_semantics=("parallel",),
        ),
    )(x, wt)
